# Initial kernel scaffold; baseline (speedup 1.0000x reference)
import jax
import jax.numpy as jnp
from jax.experimental import pallas as pl


def kernel(x, edge_index, W1, b1, W2, b2, Wfc, bfc):
    raise NotImplementedError("write your pallas kernel here")


# speedup vs baseline: 20.9799x; 20.9799x over previous
import functools

import jax
import jax.numpy as jnp
from jax import lax
from jax.experimental import pallas as pl
from jax.experimental.pallas import tpu as pltpu
from jax.experimental.pallas import tpu_sc as plsc

N = 10000
N_PAD = 10240
F_IN = 128
F_H = 64
F_OUT = 32
NC = 2
NS = 16
NW = NC * NS
CHUNK = 128
RPT = N_PAD // NS
DEG_W = 16
BLK = 2048


def _mesh():
  return plsc.VectorSubcoreMesh(
      core_axis_name="c", subcore_axis_name="s",
      num_cores=NC, num_subcores=NS)


_SC_PARAMS = pltpu.CompilerParams(use_tc_tiling_on_sc=False)


def _deg_body(n_chunks, dst_hbm, ones_hbm, zeros_hbm, out_hbm,
              idx_v, ones_v, acc_sh):
  c = lax.axis_index("c")
  s = lax.axis_index("s")
  wid = c * NS + s
  pltpu.sync_copy(zeros_hbm.at[pl.ds(s * RPT, RPT)],
                  acc_sh.at[pl.ds(s * RPT, RPT)])
  pltpu.sync_copy(ones_hbm, ones_v)
  pltpu.sync_copy(dst_hbm.at[wid], idx_v)
  plsc.subcore_barrier()

  @pl.loop(0, n_chunks)
  def _(j):
    pltpu.sync_copy(ones_v, acc_sh.at[idx_v.at[j]], add=True)

  plsc.subcore_barrier()
  pltpu.sync_copy(acc_sh.at[pl.ds(s * RPT, RPT)],
                  out_hbm.at[c, pl.ds(s * RPT, RPT)])


def _agg_body(n_chunks, src_hbm, dst_hbm, tbl_hbm, zeros_hbm, out_hbm,
              sidx_v, didx_v, rows_v, acc_sh, sem):
  c = lax.axis_index("c")
  s = lax.axis_index("s")
  wid = c * NS + s
  pltpu.sync_copy(zeros_hbm.at[pl.ds(s * RPT, RPT)],
                  acc_sh.at[pl.ds(s * RPT, RPT)])
  pltpu.sync_copy(src_hbm.at[wid], sidx_v)
  pltpu.sync_copy(dst_hbm.at[wid], didx_v)
  plsc.subcore_barrier()

  @pl.loop(0, n_chunks)
  def _(j):
    pltpu.async_copy(tbl_hbm.at[sidx_v.at[j]], rows_v, sem).wait()
    pltpu.sync_copy(rows_v, acc_sh.at[didx_v.at[j]], add=True)

  plsc.subcore_barrier()
  pltpu.sync_copy(acc_sh.at[pl.ds(s * RPT, RPT)],
                  out_hbm.at[c, pl.ds(s * RPT, RPT)])


def _make_deg_kernel(n_chunks):
  return pl.kernel(
      functools.partial(_deg_body, n_chunks),
      out_type=jax.ShapeDtypeStruct((NC, N_PAD, DEG_W), jnp.float32),
      mesh=_mesh(),
      scratch_types=[
          pltpu.VMEM((n_chunks, CHUNK), jnp.int32),
          pltpu.VMEM((CHUNK, DEG_W), jnp.float32),
          pltpu.VMEM_SHARED((N_PAD, DEG_W), jnp.float32),
      ],
      compiler_params=_SC_PARAMS,
  )


def _make_agg_kernel(n_chunks):
  return pl.kernel(
      functools.partial(_agg_body, n_chunks),
      out_type=jax.ShapeDtypeStruct((NC, N_PAD, F_H), jnp.float32),
      mesh=_mesh(),
      scratch_types=[
          pltpu.VMEM((n_chunks, CHUNK), jnp.int32),
          pltpu.VMEM((n_chunks, CHUNK), jnp.int32),
          pltpu.VMEM((CHUNK, F_H), jnp.float32),
          pltpu.VMEM_SHARED((N_PAD, F_H), jnp.float32),
          pltpu.SemaphoreType.DMA,
      ],
      compiler_params=_SC_PARAMS,
  )


def _dinv_from(degp):
  deg = degp[0, :, 0:1] + degp[1, :, 0:1] + 1.0
  return lax.rsqrt(deg)


def _tc1_body(x_ref, w_ref, degp_ref, out_ref):
  dinv = _dinv_from(degp_ref[...])
  h = jnp.dot(x_ref[...], w_ref[...], preferred_element_type=jnp.float32)
  out_ref[...] = h * dinv


def _tc2_body(q_ref, hp_ref, degp_ref, b_ref, w_ref, out_ref):
  dinv = _dinv_from(degp_ref[...])
  ssum = q_ref[0] + q_ref[1] + hp_ref[...]
  h = jnp.maximum(ssum * dinv + b_ref[...], 0.0)
  out_ref[...] = jnp.dot(
      h, w_ref[...], preferred_element_type=jnp.float32) * dinv


def _tc3_body(q_ref, hp_ref, degp_ref, b_ref, wfc_ref, bfc_ref, out_ref):
  dinv = _dinv_from(degp_ref[...])
  ssum = q_ref[0] + q_ref[1] + hp_ref[...]
  h = jnp.maximum(ssum * dinv + b_ref[...], 0.0)
  z = jnp.dot(h, wfc_ref[...], preferred_element_type=jnp.float32)
  z = z + bfc_ref[...]
  m = jnp.max(z, axis=1, keepdims=True)
  lse = jnp.log(jnp.sum(jnp.exp(z - m), axis=1, keepdims=True)) + m
  out_ref[...] = z - lse


def _row_spec(w):
  return pl.BlockSpec((BLK, w), lambda i: (i, 0))


def _full_spec(shape):
  nd = len(shape)
  return pl.BlockSpec(shape, lambda i: (0,) * nd)


_DEGP_SPEC = pl.BlockSpec((NC, BLK, DEG_W), lambda i: (0, i, 0))
_Q_SPEC = pl.BlockSpec((NC, BLK, F_H), lambda i: (0, i, 0))
_GRID = (N_PAD // BLK,)


def _tc1(x_pad, W1, degp):
  return pl.pallas_call(
      _tc1_body,
      grid=_GRID,
      in_specs=[_row_spec(F_IN), _full_spec((F_IN, F_H)), _DEGP_SPEC],
      out_specs=_row_spec(F_H),
      out_shape=jax.ShapeDtypeStruct((N_PAD, F_H), jnp.float32),
  )(x_pad, W1, degp)


def _tc2(q, hp, degp, b1, W2):
  return pl.pallas_call(
      _tc2_body,
      grid=_GRID,
      in_specs=[_Q_SPEC, _row_spec(F_H), _DEGP_SPEC,
                _full_spec((1, F_H)), _full_spec((F_H, F_H))],
      out_specs=_row_spec(F_H),
      out_shape=jax.ShapeDtypeStruct((N_PAD, F_H), jnp.float32),
  )(q, hp, degp, b1, W2)


def _tc3(q, hp, degp, b2, Wfc, bfc):
  return pl.pallas_call(
      _tc3_body,
      grid=_GRID,
      in_specs=[_Q_SPEC, _row_spec(F_H), _DEGP_SPEC,
                _full_spec((1, F_H)), _full_spec((F_H, F_OUT)),
                _full_spec((1, F_OUT))],
      out_specs=_row_spec(F_OUT),
      out_shape=jax.ShapeDtypeStruct((N_PAD, F_OUT), jnp.float32),
  )(q, hp, degp, b2, Wfc, bfc)


def kernel(x, edge_index, W1, b1, W2, b2, Wfc, bfc):
  e = edge_index.shape[1]
  src = edge_index[0].astype(jnp.int32)
  dst = edge_index[1].astype(jnp.int32)

  per_round = NW * CHUNK
  n_chunks = -(-e // per_round)
  e_pad = n_chunks * per_round
  fill = jnp.full((e_pad - e,), N, jnp.int32)
  src3 = jnp.concatenate([src, fill]).reshape(NW, n_chunks, CHUNK)
  dst3 = jnp.concatenate([dst, fill]).reshape(NW, n_chunks, CHUNK)

  x_pad = jnp.zeros((N_PAD, F_IN), jnp.float32).at[:N].set(x)
  zeros_deg = jnp.zeros((N_PAD, DEG_W), jnp.float32)
  zeros_h = jnp.zeros((N_PAD, F_H), jnp.float32)
  ones = jnp.ones((CHUNK, DEG_W), jnp.float32)

  deg_kernel = _make_deg_kernel(n_chunks)
  agg_kernel = _make_agg_kernel(n_chunks)

  degp = deg_kernel(dst3, ones, zeros_deg)
  h1p = _tc1(x_pad, W1, degp)
  q1 = agg_kernel(src3, dst3, h1p, zeros_h)
  h2p = _tc2(q1, h1p, degp, b1[None, :], W2)
  q2 = agg_kernel(src3, dst3, h2p, zeros_h)
  y = _tc3(q2, h2p, degp, b2[None, :], Wfc, bfc[None, :])
  return y[:N]


# speedup vs baseline: 26.2686x; 1.2521x over previous
import functools

import jax
import jax.numpy as jnp
from jax import lax
from jax.experimental import pallas as pl
from jax.experimental.pallas import tpu as pltpu
from jax.experimental.pallas import tpu_sc as plsc

N = 10000
N_PAD = 10240
F_IN = 128
F_H = 64
F_OUT = 32
NC = 2
NS = 16
NW = NC * NS
CHUNK = 128
RPT = N_PAD // NS
DEG_W = 16
BLK = 2048


def _mesh():
  return plsc.VectorSubcoreMesh(
      core_axis_name="c", subcore_axis_name="s",
      num_cores=NC, num_subcores=NS)


_SC_PARAMS = pltpu.CompilerParams(use_tc_tiling_on_sc=False)


def _deg_body(n_chunks, dst_hbm, ones_hbm, zeros_hbm, out_hbm,
              idx_v, ones_v, acc_sh):
  c = lax.axis_index("c")
  s = lax.axis_index("s")
  wid = c * NS + s
  pltpu.sync_copy(zeros_hbm.at[pl.ds(s * RPT, RPT)],
                  acc_sh.at[pl.ds(s * RPT, RPT)])
  pltpu.sync_copy(ones_hbm, ones_v)
  pltpu.sync_copy(dst_hbm.at[wid], idx_v)
  plsc.subcore_barrier()

  @pl.loop(0, n_chunks)
  def _(j):
    pltpu.sync_copy(ones_v, acc_sh.at[idx_v.at[j]], add=True)

  plsc.subcore_barrier()
  pltpu.sync_copy(acc_sh.at[pl.ds(s * RPT, RPT)],
                  out_hbm.at[c, pl.ds(s * RPT, RPT)])


def _agg_body(n_chunks, src_hbm, dst_hbm, tbl_hbm, zeros_hbm, out_hbm,
              sidx_v, didx_v, rows_v, acc_sh, sems):
  c = lax.axis_index("c")
  s = lax.axis_index("s")
  wid = c * NS + s
  pltpu.sync_copy(zeros_hbm.at[pl.ds(s * RPT, RPT)],
                  acc_sh.at[pl.ds(s * RPT, RPT)])
  pltpu.sync_copy(src_hbm.at[wid], sidx_v)
  pltpu.sync_copy(dst_hbm.at[wid], didx_v)
  plsc.subcore_barrier()

  def _gather(j, slot):
    return pltpu.make_async_copy(
        tbl_hbm.at[sidx_v.at[j]], rows_v.at[slot], sems.at[slot])

  _gather(0, 0).start()

  @pl.loop(0, n_chunks)
  def _(j):
    slot = lax.rem(j, 2)
    nxt = 1 - slot

    @pl.when(j + 1 < n_chunks)
    def _():
      _gather(j + 1, nxt).start()

    _gather(j, slot).wait()
    pltpu.sync_copy(rows_v.at[slot], acc_sh.at[didx_v.at[j]], add=True)

  plsc.subcore_barrier()
  pltpu.sync_copy(acc_sh.at[pl.ds(s * RPT, RPT)],
                  out_hbm.at[c, pl.ds(s * RPT, RPT)])


def _make_deg_kernel(n_chunks):
  return pl.kernel(
      functools.partial(_deg_body, n_chunks),
      out_type=jax.ShapeDtypeStruct((NC, N_PAD, DEG_W), jnp.float32),
      mesh=_mesh(),
      scratch_types=[
          pltpu.VMEM((n_chunks, CHUNK), jnp.int32),
          pltpu.VMEM((CHUNK, DEG_W), jnp.float32),
          pltpu.VMEM_SHARED((N_PAD, DEG_W), jnp.float32),
      ],
      compiler_params=_SC_PARAMS,
  )


def _make_agg_kernel(n_chunks):
  return pl.kernel(
      functools.partial(_agg_body, n_chunks),
      out_type=jax.ShapeDtypeStruct((NC, N_PAD, F_H), jnp.float32),
      mesh=_mesh(),
      scratch_types=[
          pltpu.VMEM((n_chunks, CHUNK), jnp.int32),
          pltpu.VMEM((n_chunks, CHUNK), jnp.int32),
          pltpu.VMEM((2, CHUNK, F_H), jnp.float32),
          pltpu.VMEM_SHARED((N_PAD, F_H), jnp.float32),
          pltpu.SemaphoreType.DMA((2,)),
      ],
      compiler_params=_SC_PARAMS,
  )


def _dinv_from(degp):
  deg = degp[0, :, 0:1] + degp[1, :, 0:1] + 1.0
  return lax.rsqrt(deg)


def _tc1_body(x_ref, w_ref, degp_ref, out_ref):
  dinv = _dinv_from(degp_ref[...])
  h = jnp.dot(x_ref[...], w_ref[...], preferred_element_type=jnp.float32)
  out_ref[...] = h * dinv


def _tc2_body(q_ref, hp_ref, degp_ref, b_ref, w_ref, out_ref):
  dinv = _dinv_from(degp_ref[...])
  ssum = q_ref[0] + q_ref[1] + hp_ref[...]
  h = jnp.maximum(ssum * dinv + b_ref[...], 0.0)
  out_ref[...] = jnp.dot(
      h, w_ref[...], preferred_element_type=jnp.float32) * dinv


def _tc3_body(q_ref, hp_ref, degp_ref, b_ref, wfc_ref, bfc_ref, out_ref):
  dinv = _dinv_from(degp_ref[...])
  ssum = q_ref[0] + q_ref[1] + hp_ref[...]
  h = jnp.maximum(ssum * dinv + b_ref[...], 0.0)
  z = jnp.dot(h, wfc_ref[...], preferred_element_type=jnp.float32)
  z = z + bfc_ref[...]
  m = jnp.max(z, axis=1, keepdims=True)
  lse = jnp.log(jnp.sum(jnp.exp(z - m), axis=1, keepdims=True)) + m
  out_ref[...] = z - lse


def _row_spec(w):
  return pl.BlockSpec((BLK, w), lambda i: (i, 0))


def _full_spec(shape):
  nd = len(shape)
  return pl.BlockSpec(shape, lambda i: (0,) * nd)


_DEGP_SPEC = pl.BlockSpec((NC, BLK, DEG_W), lambda i: (0, i, 0))
_Q_SPEC = pl.BlockSpec((NC, BLK, F_H), lambda i: (0, i, 0))
_GRID = (N_PAD // BLK,)


def _tc1(x_pad, W1, degp):
  return pl.pallas_call(
      _tc1_body,
      grid=_GRID,
      in_specs=[_row_spec(F_IN), _full_spec((F_IN, F_H)), _DEGP_SPEC],
      out_specs=_row_spec(F_H),
      out_shape=jax.ShapeDtypeStruct((N_PAD, F_H), jnp.float32),
  )(x_pad, W1, degp)


def _tc2(q, hp, degp, b1, W2):
  return pl.pallas_call(
      _tc2_body,
      grid=_GRID,
      in_specs=[_Q_SPEC, _row_spec(F_H), _DEGP_SPEC,
                _full_spec((1, F_H)), _full_spec((F_H, F_H))],
      out_specs=_row_spec(F_H),
      out_shape=jax.ShapeDtypeStruct((N_PAD, F_H), jnp.float32),
  )(q, hp, degp, b1, W2)


def _tc3(q, hp, degp, b2, Wfc, bfc):
  return pl.pallas_call(
      _tc3_body,
      grid=_GRID,
      in_specs=[_Q_SPEC, _row_spec(F_H), _DEGP_SPEC,
                _full_spec((1, F_H)), _full_spec((F_H, F_OUT)),
                _full_spec((1, F_OUT))],
      out_specs=_row_spec(F_OUT),
      out_shape=jax.ShapeDtypeStruct((N_PAD, F_OUT), jnp.float32),
  )(q, hp, degp, b2, Wfc, bfc)


def kernel(x, edge_index, W1, b1, W2, b2, Wfc, bfc):
  e = edge_index.shape[1]
  src = edge_index[0].astype(jnp.int32)
  dst = edge_index[1].astype(jnp.int32)

  per_round = NW * CHUNK
  n_chunks = -(-e // per_round)
  e_pad = n_chunks * per_round
  fill = jnp.full((e_pad - e,), N, jnp.int32)
  src3 = jnp.concatenate([src, fill]).reshape(NW, n_chunks, CHUNK)
  dst3 = jnp.concatenate([dst, fill]).reshape(NW, n_chunks, CHUNK)

  x_pad = jnp.zeros((N_PAD, F_IN), jnp.float32).at[:N].set(x)
  zeros_deg = jnp.zeros((N_PAD, DEG_W), jnp.float32)
  zeros_h = jnp.zeros((N_PAD, F_H), jnp.float32)
  ones = jnp.ones((CHUNK, DEG_W), jnp.float32)

  deg_kernel = _make_deg_kernel(n_chunks)
  agg_kernel = _make_agg_kernel(n_chunks)

  degp = deg_kernel(dst3, ones, zeros_deg)
  h1p = _tc1(x_pad, W1, degp)
  q1 = agg_kernel(src3, dst3, h1p, zeros_h)
  h2p = _tc2(q1, h1p, degp, b1[None, :], W2)
  q2 = agg_kernel(src3, dst3, h2p, zeros_h)
  y = _tc3(q2, h2p, degp, b2[None, :], Wfc, bfc[None, :])
  return y[:N]


# speedup vs baseline: 34.9466x; 1.3304x over previous
import functools

import jax
import jax.numpy as jnp
from jax import lax
from jax.experimental import pallas as pl
from jax.experimental.pallas import tpu as pltpu
from jax.experimental.pallas import tpu_sc as plsc

N = 10000
N_PAD = 10240
F_IN = 128
F_H = 64
F_OUT = 32
NC = 2
NS = 16
NW = NC * NS
CHUNK = 125
RPT = N_PAD // NS
DEG_W = 16
BLK = 2048


def _mesh():
  return plsc.VectorSubcoreMesh(
      core_axis_name="c", subcore_axis_name="s",
      num_cores=NC, num_subcores=NS)


_SC_PARAMS = pltpu.CompilerParams(use_tc_tiling_on_sc=False)


def _deg_body(n_chunks, dst_hbm, ones_hbm, zeros_hbm, out_hbm,
              idx_v, ones_v, acc_sh, sems):
  c = lax.axis_index("c")
  s = lax.axis_index("s")
  wid = c * NS + s
  pltpu.sync_copy(zeros_hbm.at[pl.ds(s * RPT, RPT)],
                  acc_sh.at[pl.ds(s * RPT, RPT)])
  pltpu.sync_copy(ones_hbm, ones_v)
  pltpu.sync_copy(dst_hbm.at[wid], idx_v)
  plsc.subcore_barrier()

  def _scat(j):
    return pltpu.async_copy(
        ones_v, acc_sh.at[idx_v.at[j]], sems.at[lax.rem(j, 2)], add=True)

  def _wait(j):
    pltpu.make_async_copy(
        ones_v, acc_sh.at[idx_v.at[j]], sems.at[lax.rem(j, 2)]).wait()

  _scat(0)

  @pl.loop(1, n_chunks)
  def _(j):
    _scat(j)
    _wait(j - 1)

  _wait(n_chunks - 1)
  plsc.subcore_barrier()
  pltpu.sync_copy(acc_sh.at[pl.ds(s * RPT, RPT)],
                  out_hbm.at[c, pl.ds(s * RPT, RPT)])


def _agg_body(n_chunks, src_hbm, dst_hbm, tbl_hbm, zeros_hbm, out_hbm,
              sidx_v, didx_v, rows_v, tbl_sh, acc_sh, sems):
  c = lax.axis_index("c")
  s = lax.axis_index("s")
  wid = c * NS + s
  pltpu.sync_copy(zeros_hbm.at[pl.ds(s * RPT, RPT)],
                  acc_sh.at[pl.ds(s * RPT, RPT)])
  pltpu.sync_copy(tbl_hbm.at[pl.ds(s * RPT, RPT)],
                  tbl_sh.at[pl.ds(s * RPT, RPT)])
  pltpu.sync_copy(src_hbm.at[wid], sidx_v)
  pltpu.sync_copy(dst_hbm.at[wid], didx_v)
  plsc.subcore_barrier()

  def _gather(j, slot):
    return pltpu.make_async_copy(
        tbl_sh.at[sidx_v.at[j]], rows_v.at[slot], sems.at[slot])

  _gather(0, 0).start()

  @pl.loop(0, n_chunks)
  def _(j):
    slot = lax.rem(j, 2)
    nxt = 1 - slot

    @pl.when(j + 1 < n_chunks)
    def _():
      _gather(j + 1, nxt).start()

    _gather(j, slot).wait()
    pltpu.sync_copy(rows_v.at[slot], acc_sh.at[didx_v.at[j]], add=True)

  plsc.subcore_barrier()
  pltpu.sync_copy(acc_sh.at[pl.ds(s * RPT, RPT)],
                  out_hbm.at[c, pl.ds(s * RPT, RPT)])


def _make_deg_kernel(n_chunks):
  return pl.kernel(
      functools.partial(_deg_body, n_chunks),
      out_type=jax.ShapeDtypeStruct((NC, N_PAD, DEG_W), jnp.float32),
      mesh=_mesh(),
      scratch_types=[
          pltpu.VMEM((n_chunks, CHUNK), jnp.int32),
          pltpu.VMEM((CHUNK, DEG_W), jnp.float32),
          pltpu.VMEM_SHARED((N_PAD, DEG_W), jnp.float32),
          pltpu.SemaphoreType.DMA((2,)),
      ],
      compiler_params=_SC_PARAMS,
  )


def _make_agg_kernel(n_chunks):
  return pl.kernel(
      functools.partial(_agg_body, n_chunks),
      out_type=jax.ShapeDtypeStruct((NC, N_PAD, F_H), jnp.float32),
      mesh=_mesh(),
      scratch_types=[
          pltpu.VMEM((n_chunks, CHUNK), jnp.int32),
          pltpu.VMEM((n_chunks, CHUNK), jnp.int32),
          pltpu.VMEM((2, CHUNK, F_H), jnp.float32),
          pltpu.VMEM_SHARED((N_PAD, F_H), jnp.float32),
          pltpu.VMEM_SHARED((N_PAD, F_H), jnp.float32),
          pltpu.SemaphoreType.DMA((2,)),
      ],
      compiler_params=_SC_PARAMS,
  )


def _dinv_from(degp):
  deg = degp[0, :, 0:1] + degp[1, :, 0:1] + 1.0
  return lax.rsqrt(deg)


def _tc1_body(x_ref, w_ref, degp_ref, out_ref):
  dinv = _dinv_from(degp_ref[...])
  h = jnp.dot(x_ref[...], w_ref[...], preferred_element_type=jnp.float32)
  out_ref[...] = h * dinv


def _tc2_body(q_ref, hp_ref, degp_ref, b_ref, w_ref, out_ref):
  dinv = _dinv_from(degp_ref[...])
  ssum = q_ref[0] + q_ref[1] + hp_ref[...]
  h = jnp.maximum(ssum * dinv + b_ref[...], 0.0)
  out_ref[...] = jnp.dot(
      h, w_ref[...], preferred_element_type=jnp.float32) * dinv


def _tc3_body(q_ref, hp_ref, degp_ref, b_ref, wfc_ref, bfc_ref, out_ref):
  dinv = _dinv_from(degp_ref[...])
  ssum = q_ref[0] + q_ref[1] + hp_ref[...]
  h = jnp.maximum(ssum * dinv + b_ref[...], 0.0)
  z = jnp.dot(h, wfc_ref[...], preferred_element_type=jnp.float32)
  z = z + bfc_ref[...]
  m = jnp.max(z, axis=1, keepdims=True)
  lse = jnp.log(jnp.sum(jnp.exp(z - m), axis=1, keepdims=True)) + m
  out_ref[...] = z - lse


def _row_spec(w):
  return pl.BlockSpec((BLK, w), lambda i: (i, 0))


def _full_spec(shape):
  nd = len(shape)
  return pl.BlockSpec(shape, lambda i: (0,) * nd)


_DEGP_SPEC = pl.BlockSpec((NC, BLK, DEG_W), lambda i: (0, i, 0))
_Q_SPEC = pl.BlockSpec((NC, BLK, F_H), lambda i: (0, i, 0))
_GRID = (N_PAD // BLK,)


def _tc1(x_pad, W1, degp):
  return pl.pallas_call(
      _tc1_body,
      grid=_GRID,
      in_specs=[_row_spec(F_IN), _full_spec((F_IN, F_H)), _DEGP_SPEC],
      out_specs=_row_spec(F_H),
      out_shape=jax.ShapeDtypeStruct((N_PAD, F_H), jnp.float32),
  )(x_pad, W1, degp)


def _tc2(q, hp, degp, b1, W2):
  return pl.pallas_call(
      _tc2_body,
      grid=_GRID,
      in_specs=[_Q_SPEC, _row_spec(F_H), _DEGP_SPEC,
                _full_spec((1, F_H)), _full_spec((F_H, F_H))],
      out_specs=_row_spec(F_H),
      out_shape=jax.ShapeDtypeStruct((N_PAD, F_H), jnp.float32),
  )(q, hp, degp, b1, W2)


def _tc3(q, hp, degp, b2, Wfc, bfc):
  return pl.pallas_call(
      _tc3_body,
      grid=_GRID,
      in_specs=[_Q_SPEC, _row_spec(F_H), _DEGP_SPEC,
                _full_spec((1, F_H)), _full_spec((F_H, F_OUT)),
                _full_spec((1, F_OUT))],
      out_specs=_row_spec(F_OUT),
      out_shape=jax.ShapeDtypeStruct((N, F_OUT), jnp.float32),
  )(q, hp, degp, b2, Wfc, bfc)


def kernel(x, edge_index, W1, b1, W2, b2, Wfc, bfc):
  e = edge_index.shape[1]
  src = edge_index[0].astype(jnp.int32)
  dst = edge_index[1].astype(jnp.int32)

  per_round = NW * CHUNK
  n_chunks = -(-e // per_round)
  e_pad = n_chunks * per_round
  if e_pad != e:
    fill = jnp.full((e_pad - e,), N, jnp.int32)
    src = jnp.concatenate([src, fill])
    dst = jnp.concatenate([dst, fill])
  src3 = src.reshape(NW, n_chunks, CHUNK)
  dst3 = dst.reshape(NW, n_chunks, CHUNK)

  x_pad = jnp.zeros((N_PAD, F_IN), jnp.float32).at[:N].set(x)
  zeros_deg = jnp.zeros((N_PAD, DEG_W), jnp.float32)
  zeros_h = jnp.zeros((N_PAD, F_H), jnp.float32)
  ones = jnp.ones((CHUNK, DEG_W), jnp.float32)

  deg_kernel = _make_deg_kernel(n_chunks)
  agg_kernel = _make_agg_kernel(n_chunks)

  degp = deg_kernel(dst3, ones, zeros_deg)
  h1p = _tc1(x_pad, W1, degp)
  q1 = agg_kernel(src3, dst3, h1p, zeros_h)
  h2p = _tc2(q1, h1p, degp, b1[None, :], W2)
  q2 = agg_kernel(src3, dst3, h2p, zeros_h)
  return _tc3(q2, h2p, degp, b2[None, :], Wfc, bfc[None, :])


# speedup vs baseline: 39.6717x; 1.1352x over previous
import functools

import jax
import jax.numpy as jnp
from jax import lax
from jax.experimental import pallas as pl
from jax.experimental.pallas import tpu as pltpu
from jax.experimental.pallas import tpu_sc as plsc

N = 10000
N_PAD = 10240
F_IN = 128
F_H = 64
F_OUT = 32
NC = 2
NS = 16
NW = NC * NS
CHUNK = 125
RPT = N_PAD // NS
DEG_W = 16
BLK = 2048


def _mesh():
  return plsc.VectorSubcoreMesh(
      core_axis_name="c", subcore_axis_name="s",
      num_cores=NC, num_subcores=NS)


_SC_PARAMS = pltpu.CompilerParams(use_tc_tiling_on_sc=False)


def _deg_body(n_chunks, dst_hbm, ones_hbm, zeros_hbm, out_hbm,
              idx_v, ones_v, acc_sh, sems):
  c = lax.axis_index("c")
  s = lax.axis_index("s")
  wid = c * NS + s
  pltpu.sync_copy(zeros_hbm.at[pl.ds(s * RPT, RPT)],
                  acc_sh.at[pl.ds(s * RPT, RPT)])
  pltpu.sync_copy(ones_hbm, ones_v)
  pltpu.sync_copy(dst_hbm.at[wid], idx_v)
  plsc.subcore_barrier()

  def _scat(j):
    return pltpu.async_copy(
        ones_v, acc_sh.at[idx_v.at[j]], sems.at[lax.rem(j, 2)], add=True)

  def _wait(j):
    pltpu.make_async_copy(
        ones_v, acc_sh.at[idx_v.at[j]], sems.at[lax.rem(j, 2)]).wait()

  _scat(0)

  @pl.loop(1, n_chunks)
  def _(j):
    _scat(j)
    _wait(j - 1)

  _wait(n_chunks - 1)
  plsc.subcore_barrier()
  pltpu.sync_copy(acc_sh.at[pl.ds(s * RPT, RPT)],
                  out_hbm.at[c, pl.ds(s * RPT, RPT)])


def _agg_body(n_chunks, src_hbm, dst_hbm, tbl_hbm, zeros_hbm, out_hbm,
              sidx_v, didx_v, rows_v, tbl_sh, acc_sh, gsems, ssems, psems):
  c = lax.axis_index("c")
  s = lax.axis_index("s")
  wid = c * NS + s
  pltpu.async_copy(zeros_hbm.at[pl.ds(s * RPT, RPT)],
                   acc_sh.at[pl.ds(s * RPT, RPT)], psems.at[0])
  pltpu.async_copy(tbl_hbm.at[pl.ds(s * RPT, RPT)],
                   tbl_sh.at[pl.ds(s * RPT, RPT)], psems.at[1])
  pltpu.sync_copy(src_hbm.at[wid], sidx_v)
  pltpu.sync_copy(dst_hbm.at[wid], didx_v)
  pltpu.make_async_copy(zeros_hbm.at[pl.ds(s * RPT, RPT)],
                        acc_sh.at[pl.ds(s * RPT, RPT)], psems.at[0]).wait()
  pltpu.make_async_copy(tbl_hbm.at[pl.ds(s * RPT, RPT)],
                        tbl_sh.at[pl.ds(s * RPT, RPT)], psems.at[1]).wait()
  plsc.subcore_barrier()

  def _gather(j, slot):
    return pltpu.make_async_copy(
        tbl_sh.at[sidx_v.at[j]], rows_v.at[slot], gsems.at[slot])

  def _scat_start(j, slot):
    pltpu.async_copy(rows_v.at[slot], acc_sh.at[didx_v.at[j]],
                     ssems.at[slot], add=True)

  def _scat_wait(j, slot):
    pltpu.make_async_copy(rows_v.at[slot], acc_sh.at[didx_v.at[j]],
                          ssems.at[slot]).wait()

  _gather(0, 0).start()
  _gather(1, 1).start()

  @pl.loop(0, n_chunks)
  def _(j):
    slot = lax.rem(j, 3)
    _gather(j, slot).wait()
    _scat_start(j, slot)
    nj = j + 2

    @pl.when(nj < n_chunks)
    def _():
      ns = lax.rem(nj, 3)

      @pl.when(j >= 1)
      def _():
        _scat_wait(j - 1, ns)

      _gather(nj, ns).start()

  for t in range(3):
    j = n_chunks - 3 + t
    _scat_wait(j, j % 3)
  plsc.subcore_barrier()
  pltpu.sync_copy(acc_sh.at[pl.ds(s * RPT, RPT)],
                  out_hbm.at[c, pl.ds(s * RPT, RPT)])


def _make_deg_kernel(n_chunks):
  return pl.kernel(
      functools.partial(_deg_body, n_chunks),
      out_type=jax.ShapeDtypeStruct((NC, N_PAD, DEG_W), jnp.float32),
      mesh=_mesh(),
      scratch_types=[
          pltpu.VMEM((n_chunks, CHUNK), jnp.int32),
          pltpu.VMEM((CHUNK, DEG_W), jnp.float32),
          pltpu.VMEM_SHARED((N_PAD, DEG_W), jnp.float32),
          pltpu.SemaphoreType.DMA((2,)),
      ],
      compiler_params=_SC_PARAMS,
  )


def _make_agg_kernel(n_chunks):
  return pl.kernel(
      functools.partial(_agg_body, n_chunks),
      out_type=jax.ShapeDtypeStruct((NC, N_PAD, F_H), jnp.float32),
      mesh=_mesh(),
      scratch_types=[
          pltpu.VMEM((n_chunks, CHUNK), jnp.int32),
          pltpu.VMEM((n_chunks, CHUNK), jnp.int32),
          pltpu.VMEM((3, CHUNK, F_H), jnp.float32),
          pltpu.VMEM_SHARED((N_PAD, F_H), jnp.float32),
          pltpu.VMEM_SHARED((N_PAD, F_H), jnp.float32),
          pltpu.SemaphoreType.DMA((3,)),
          pltpu.SemaphoreType.DMA((3,)),
          pltpu.SemaphoreType.DMA((2,)),
      ],
      compiler_params=_SC_PARAMS,
  )


def _dinv_from(degp):
  deg = degp[0, :, 0:1] + degp[1, :, 0:1] + 1.0
  return lax.rsqrt(deg)


def _tc1_body(x_ref, w_ref, degp_ref, out_ref):
  dinv = _dinv_from(degp_ref[...])
  h = jnp.dot(x_ref[...], w_ref[...], preferred_element_type=jnp.float32)
  out_ref[...] = h * dinv


def _tc2_body(q_ref, hp_ref, degp_ref, b_ref, w_ref, out_ref):
  dinv = _dinv_from(degp_ref[...])
  ssum = q_ref[0] + q_ref[1] + hp_ref[...]
  h = jnp.maximum(ssum * dinv + b_ref[...], 0.0)
  out_ref[...] = jnp.dot(
      h, w_ref[...], preferred_element_type=jnp.float32) * dinv


def _tc3_body(q_ref, hp_ref, degp_ref, b_ref, wfc_ref, bfc_ref, out_ref):
  dinv = _dinv_from(degp_ref[...])
  ssum = q_ref[0] + q_ref[1] + hp_ref[...]
  h = jnp.maximum(ssum * dinv + b_ref[...], 0.0)
  z = jnp.dot(h, wfc_ref[...], preferred_element_type=jnp.float32)
  z = z + bfc_ref[...]
  m = jnp.max(z, axis=1, keepdims=True)
  lse = jnp.log(jnp.sum(jnp.exp(z - m), axis=1, keepdims=True)) + m
  out_ref[...] = z - lse


def _row_spec(w):
  return pl.BlockSpec((BLK, w), lambda i: (i, 0))


def _full_spec(shape):
  nd = len(shape)
  return pl.BlockSpec(shape, lambda i: (0,) * nd)


_DEGP_SPEC = pl.BlockSpec((NC, BLK, DEG_W), lambda i: (0, i, 0))
_Q_SPEC = pl.BlockSpec((NC, BLK, F_H), lambda i: (0, i, 0))
_GRID = (N_PAD // BLK,)


def _tc1(x_pad, W1, degp):
  return pl.pallas_call(
      _tc1_body,
      grid=_GRID,
      in_specs=[_row_spec(F_IN), _full_spec((F_IN, F_H)), _DEGP_SPEC],
      out_specs=_row_spec(F_H),
      out_shape=jax.ShapeDtypeStruct((N_PAD, F_H), jnp.float32),
  )(x_pad, W1, degp)


def _tc2(q, hp, degp, b1, W2):
  return pl.pallas_call(
      _tc2_body,
      grid=_GRID,
      in_specs=[_Q_SPEC, _row_spec(F_H), _DEGP_SPEC,
                _full_spec((1, F_H)), _full_spec((F_H, F_H))],
      out_specs=_row_spec(F_H),
      out_shape=jax.ShapeDtypeStruct((N_PAD, F_H), jnp.float32),
  )(q, hp, degp, b1, W2)


def _tc3(q, hp, degp, b2, Wfc, bfc):
  return pl.pallas_call(
      _tc3_body,
      grid=_GRID,
      in_specs=[_Q_SPEC, _row_spec(F_H), _DEGP_SPEC,
                _full_spec((1, F_H)), _full_spec((F_H, F_OUT)),
                _full_spec((1, F_OUT))],
      out_specs=_row_spec(F_OUT),
      out_shape=jax.ShapeDtypeStruct((N, F_OUT), jnp.float32),
  )(q, hp, degp, b2, Wfc, bfc)


def kernel(x, edge_index, W1, b1, W2, b2, Wfc, bfc):
  e = edge_index.shape[1]
  src = edge_index[0].astype(jnp.int32)
  dst = edge_index[1].astype(jnp.int32)

  per_round = NW * CHUNK
  n_chunks = -(-e // per_round)
  e_pad = n_chunks * per_round
  if e_pad != e:
    fill = jnp.full((e_pad - e,), N, jnp.int32)
    src = jnp.concatenate([src, fill])
    dst = jnp.concatenate([dst, fill])
  src3 = src.reshape(NW, n_chunks, CHUNK)
  dst3 = dst.reshape(NW, n_chunks, CHUNK)

  x_pad = jnp.zeros((N_PAD, F_IN), jnp.float32).at[:N].set(x)
  zeros_deg = jnp.zeros((N_PAD, DEG_W), jnp.float32)
  zeros_h = jnp.zeros((N_PAD, F_H), jnp.float32)
  ones = jnp.ones((CHUNK, DEG_W), jnp.float32)

  deg_kernel = _make_deg_kernel(n_chunks)
  agg_kernel = _make_agg_kernel(n_chunks)

  degp = deg_kernel(dst3, ones, zeros_deg)
  h1p = _tc1(x_pad, W1, degp)
  q1 = agg_kernel(src3, dst3, h1p, zeros_h)
  h2p = _tc2(q1, h1p, degp, b1[None, :], W2)
  q2 = agg_kernel(src3, dst3, h2p, zeros_h)
  return _tc3(q2, h2p, degp, b2[None, :], Wfc, bfc[None, :])


# speedup vs baseline: 39.7876x; 1.0029x over previous
import functools

import jax
import jax.numpy as jnp
from jax import lax
from jax.experimental import pallas as pl
from jax.experimental.pallas import tpu as pltpu
from jax.experimental.pallas import tpu_sc as plsc

N = 10000
N_PAD = 10240
F_IN = 128
F_H = 64
F_OUT = 32
NC = 2
NS = 16
NW = NC * NS
CHUNK = 125
RPT = N_PAD // NS
DEG_W = 16
BLK = 2000


def _mesh():
  return plsc.VectorSubcoreMesh(
      core_axis_name="c", subcore_axis_name="s",
      num_cores=NC, num_subcores=NS)


_SC_PARAMS = pltpu.CompilerParams(use_tc_tiling_on_sc=False)


def _deg_body(n_chunks, eidx_hbm, ones_hbm, zeros_hbm, out_hbm,
              idx_v, ones_v, acc_sh, sems):
  c = lax.axis_index("c")
  s = lax.axis_index("s")
  wid = c * NS + s
  pltpu.sync_copy(zeros_hbm.at[pl.ds(s * RPT, RPT)],
                  acc_sh.at[pl.ds(s * RPT, RPT)])
  pltpu.sync_copy(ones_hbm, ones_v)
  pltpu.sync_copy(eidx_hbm.at[1, wid], idx_v)
  plsc.subcore_barrier()

  def _scat(j):
    return pltpu.async_copy(
        ones_v, acc_sh.at[idx_v.at[j]], sems.at[lax.rem(j, 2)], add=True)

  def _wait(j):
    pltpu.make_async_copy(
        ones_v, acc_sh.at[idx_v.at[j]], sems.at[lax.rem(j, 2)]).wait()

  _scat(0)

  @pl.loop(1, n_chunks)
  def _(j):
    _scat(j)
    _wait(j - 1)

  _wait(n_chunks - 1)
  plsc.subcore_barrier()
  pltpu.sync_copy(acc_sh.at[pl.ds(s * RPT, RPT)],
                  out_hbm.at[c, pl.ds(s * RPT, RPT)])


def _agg_body(n_chunks, eidx_hbm, tbl_hbm, zeros_hbm, out_hbm,
              sidx_v, didx_v, rows_v, tbl_sh, acc_sh, gsems, ssems, psems):
  c = lax.axis_index("c")
  s = lax.axis_index("s")
  wid = c * NS + s
  pltpu.async_copy(zeros_hbm.at[pl.ds(s * RPT, RPT)],
                   acc_sh.at[pl.ds(s * RPT, RPT)], psems.at[0])
  pltpu.async_copy(tbl_hbm.at[pl.ds(s * RPT, RPT)],
                   tbl_sh.at[pl.ds(s * RPT, RPT)], psems.at[1])
  pltpu.sync_copy(eidx_hbm.at[0, wid], sidx_v)
  pltpu.sync_copy(eidx_hbm.at[1, wid], didx_v)
  pltpu.make_async_copy(zeros_hbm.at[pl.ds(s * RPT, RPT)],
                        acc_sh.at[pl.ds(s * RPT, RPT)], psems.at[0]).wait()
  pltpu.make_async_copy(tbl_hbm.at[pl.ds(s * RPT, RPT)],
                        tbl_sh.at[pl.ds(s * RPT, RPT)], psems.at[1]).wait()
  plsc.subcore_barrier()

  def _gather(j, slot):
    return pltpu.make_async_copy(
        tbl_sh.at[sidx_v.at[j]], rows_v.at[slot], gsems.at[slot])

  def _scat_start(j, slot):
    pltpu.async_copy(rows_v.at[slot], acc_sh.at[didx_v.at[j]],
                     ssems.at[slot], add=True)

  def _scat_wait(j, slot):
    pltpu.make_async_copy(rows_v.at[slot], acc_sh.at[didx_v.at[j]],
                          ssems.at[slot]).wait()

  _gather(0, 0).start()
  _gather(1, 1).start()

  @pl.loop(0, n_chunks)
  def _(j):
    slot = lax.rem(j, 3)
    _gather(j, slot).wait()
    _scat_start(j, slot)
    nj = j + 2

    @pl.when(nj < n_chunks)
    def _():
      ns = lax.rem(nj, 3)

      @pl.when(j >= 1)
      def _():
        _scat_wait(j - 1, ns)

      _gather(nj, ns).start()

  for t in range(3):
    j = n_chunks - 3 + t
    _scat_wait(j, j % 3)
  plsc.subcore_barrier()
  pltpu.sync_copy(acc_sh.at[pl.ds(s * RPT, RPT)],
                  out_hbm.at[c, pl.ds(s * RPT, RPT)])


def _make_deg_kernel(n_chunks):
  return pl.kernel(
      functools.partial(_deg_body, n_chunks),
      out_type=jax.ShapeDtypeStruct((NC, N_PAD, DEG_W), jnp.float32),
      mesh=_mesh(),
      scratch_types=[
          pltpu.VMEM((n_chunks, CHUNK), jnp.int32),
          pltpu.VMEM((CHUNK, DEG_W), jnp.float32),
          pltpu.VMEM_SHARED((N_PAD, DEG_W), jnp.float32),
          pltpu.SemaphoreType.DMA((2,)),
      ],
      compiler_params=_SC_PARAMS,
  )


def _make_agg_kernel(n_chunks):
  return pl.kernel(
      functools.partial(_agg_body, n_chunks),
      out_type=jax.ShapeDtypeStruct((NC, N_PAD, F_H), jnp.float32),
      mesh=_mesh(),
      scratch_types=[
          pltpu.VMEM((n_chunks, CHUNK), jnp.int32),
          pltpu.VMEM((n_chunks, CHUNK), jnp.int32),
          pltpu.VMEM((3, CHUNK, F_H), jnp.float32),
          pltpu.VMEM_SHARED((N_PAD, F_H), jnp.float32),
          pltpu.VMEM_SHARED((N_PAD, F_H), jnp.float32),
          pltpu.SemaphoreType.DMA((3,)),
          pltpu.SemaphoreType.DMA((3,)),
          pltpu.SemaphoreType.DMA((2,)),
      ],
      compiler_params=_SC_PARAMS,
  )


def _dinv_from(degp):
  deg = degp[0, :, 0:1] + degp[1, :, 0:1] + 1.0
  return lax.rsqrt(deg)


def _tc1_body(x_ref, w_ref, degp_ref, out_ref):
  dinv = _dinv_from(degp_ref[...])
  h = jnp.dot(x_ref[...], w_ref[...], preferred_element_type=jnp.float32)
  out_ref[...] = h * dinv


def _tc2_body(q_ref, hp_ref, degp_ref, b_ref, w_ref, out_ref):
  dinv = _dinv_from(degp_ref[...])
  ssum = q_ref[...] + hp_ref[...]
  h = jnp.maximum(ssum * dinv + b_ref[...], 0.0)
  out_ref[...] = jnp.dot(
      h, w_ref[...], preferred_element_type=jnp.float32) * dinv


def _tc3_body(q_ref, hp_ref, degp_ref, b_ref, wfc_ref, bfc_ref, out_ref):
  dinv = _dinv_from(degp_ref[...])
  ssum = q_ref[...] + hp_ref[...]
  h = jnp.maximum(ssum * dinv + b_ref[...], 0.0)
  z = jnp.dot(h, wfc_ref[...], preferred_element_type=jnp.float32)
  z = z + bfc_ref[...]
  m = jnp.max(z, axis=1, keepdims=True)
  lse = jnp.log(jnp.sum(jnp.exp(z - m), axis=1, keepdims=True)) + m
  out_ref[...] = z - lse


def _row_spec(w):
  return pl.BlockSpec((BLK, w), lambda i: (i, 0))


def _full_spec(shape):
  nd = len(shape)
  return pl.BlockSpec(shape, lambda i: (0,) * nd)


_DEGP_SPEC = pl.BlockSpec((NC, BLK, DEG_W), lambda i: (0, i, 0))
_GRID = (N // BLK,)


def _tc1(x, W1, degp):
  return pl.pallas_call(
      _tc1_body,
      grid=_GRID,
      in_specs=[_row_spec(F_IN), _full_spec((F_IN, F_H)), _DEGP_SPEC],
      out_specs=_row_spec(F_H),
      out_shape=jax.ShapeDtypeStruct((N_PAD, F_H), jnp.float32),
  )(x, W1, degp)


def _tc2(q, hp, degp, b1, W2):
  return pl.pallas_call(
      _tc2_body,
      grid=_GRID,
      in_specs=[_row_spec(F_H), _row_spec(F_H), _DEGP_SPEC,
                _full_spec((1, F_H)), _full_spec((F_H, F_H))],
      out_specs=_row_spec(F_H),
      out_shape=jax.ShapeDtypeStruct((N_PAD, F_H), jnp.float32),
  )(q, hp, degp, b1, W2)


def _tc3(q, hp, degp, b2, Wfc, bfc):
  return pl.pallas_call(
      _tc3_body,
      grid=_GRID,
      in_specs=[_row_spec(F_H), _row_spec(F_H), _DEGP_SPEC,
                _full_spec((1, F_H)), _full_spec((F_H, F_OUT)),
                _full_spec((1, F_OUT))],
      out_specs=_row_spec(F_OUT),
      out_shape=jax.ShapeDtypeStruct((N, F_OUT), jnp.float32),
  )(q, hp, degp, b2, Wfc, bfc)


def kernel(x, edge_index, W1, b1, W2, b2, Wfc, bfc):
  e = edge_index.shape[1]
  ei = edge_index.astype(jnp.int32)

  per_round = NW * CHUNK
  n_chunks = -(-e // per_round)
  e_pad = n_chunks * per_round
  if e_pad != e:
    fill = jnp.full((2, e_pad - e), N, jnp.int32)
    ei = jnp.concatenate([ei, fill], axis=1)
  eidx = ei.reshape(2, NW, n_chunks, CHUNK)

  zeros_deg = jnp.zeros((N_PAD, DEG_W), jnp.float32)
  zeros_h = jnp.zeros((N_PAD, F_H), jnp.float32)
  ones = jnp.ones((CHUNK, DEG_W), jnp.float32)

  deg_kernel = _make_deg_kernel(n_chunks)
  agg_kernel = _make_agg_kernel(n_chunks)

  degp = deg_kernel(eidx, ones, zeros_deg)
  h1p = _tc1(x, W1, degp)
  q1 = agg_kernel(eidx, h1p, zeros_h)
  h2p = _tc2(q1[0] + q1[1], h1p, degp, b1[None, :], W2)
  q2 = agg_kernel(eidx, h2p, zeros_h)
  return _tc3(q2[0] + q2[1], h2p, degp, b2[None, :], Wfc, bfc[None, :])


# speedup vs baseline: 41.3878x; 1.0402x over previous
import functools

import jax
import jax.numpy as jnp
from jax import lax
from jax.experimental import pallas as pl
from jax.experimental.pallas import tpu as pltpu
from jax.experimental.pallas import tpu_sc as plsc

N = 10000
N_PAD = 10240
F_IN = 128
F_H = 64
F_OUT = 32
NC = 2
NS = 16
NW = NC * NS
CHUNK = 125
RPT = N_PAD // NS
DEG_W = 16
BLK = 2000


def _mesh():
  return plsc.VectorSubcoreMesh(
      core_axis_name="c", subcore_axis_name="s",
      num_cores=NC, num_subcores=NS)


_SC_PARAMS = pltpu.CompilerParams(use_tc_tiling_on_sc=False)


def _deg_body(n_chunks, eidx_hbm, ones_hbm, zeros_hbm, out_hbm,
              idx_v, ones_v, acc_sh, sems):
  c = lax.axis_index("c")
  s = lax.axis_index("s")
  wid = c * NS + s
  pltpu.sync_copy(zeros_hbm.at[pl.ds(s * RPT, RPT)],
                  acc_sh.at[pl.ds(s * RPT, RPT)])
  pltpu.sync_copy(ones_hbm, ones_v)
  pltpu.sync_copy(eidx_hbm.at[1, wid], idx_v)
  plsc.subcore_barrier()

  def _scat(j):
    return pltpu.async_copy(
        ones_v, acc_sh.at[idx_v.at[j]], sems.at[lax.rem(j, 2)], add=True)

  def _wait(j):
    pltpu.make_async_copy(
        ones_v, acc_sh.at[idx_v.at[j]], sems.at[lax.rem(j, 2)]).wait()

  _scat(0)

  @pl.loop(1, n_chunks)
  def _(j):
    _scat(j)
    _wait(j - 1)

  _wait(n_chunks - 1)
  plsc.subcore_barrier()
  pltpu.sync_copy(acc_sh.at[pl.ds(s * RPT, RPT)],
                  out_hbm.at[c, pl.ds(s * RPT, RPT)])


def _agg_body(n_chunks, eidx_hbm, tbl_hbm, zeros_hbm, out_hbm,
              sidx_v, didx_v, rows_v, tbl_sh, acc_sh, gsems, ssems, psems):
  c = lax.axis_index("c")
  s = lax.axis_index("s")
  wid = c * NS + s
  pltpu.async_copy(zeros_hbm.at[pl.ds(s * RPT, RPT)],
                   acc_sh.at[pl.ds(s * RPT, RPT)], psems.at[0])
  pltpu.async_copy(tbl_hbm.at[pl.ds(s * RPT, RPT)],
                   tbl_sh.at[pl.ds(s * RPT, RPT)], psems.at[1])
  pltpu.sync_copy(eidx_hbm.at[0, wid], sidx_v)
  pltpu.sync_copy(eidx_hbm.at[1, wid], didx_v)
  pltpu.make_async_copy(zeros_hbm.at[pl.ds(s * RPT, RPT)],
                        acc_sh.at[pl.ds(s * RPT, RPT)], psems.at[0]).wait()
  pltpu.make_async_copy(tbl_hbm.at[pl.ds(s * RPT, RPT)],
                        tbl_sh.at[pl.ds(s * RPT, RPT)], psems.at[1]).wait()
  plsc.subcore_barrier()

  def _gather(j, slot):
    return pltpu.make_async_copy(
        tbl_sh.at[sidx_v.at[j]], rows_v.at[slot], gsems.at[slot])

  def _scat_start(j, slot):
    pltpu.async_copy(rows_v.at[slot], acc_sh.at[didx_v.at[j]],
                     ssems.at[slot], add=True)

  def _scat_wait(j, slot):
    pltpu.make_async_copy(rows_v.at[slot], acc_sh.at[didx_v.at[j]],
                          ssems.at[slot]).wait()

  _gather(0, 0).start()
  _gather(1, 1).start()

  @pl.loop(0, n_chunks)
  def _(j):
    slot = lax.rem(j, 3)
    _gather(j, slot).wait()
    _scat_start(j, slot)
    nj = j + 2

    @pl.when(nj < n_chunks)
    def _():
      ns = lax.rem(nj, 3)

      @pl.when(j >= 1)
      def _():
        _scat_wait(j - 1, ns)

      _gather(nj, ns).start()

  for t in range(3):
    j = n_chunks - 3 + t
    _scat_wait(j, j % 3)
  plsc.subcore_barrier()
  pltpu.sync_copy(acc_sh.at[pl.ds(s * RPT, RPT)],
                  out_hbm.at[c, pl.ds(s * RPT, RPT)])


def _make_deg_kernel(n_chunks):
  return pl.kernel(
      functools.partial(_deg_body, n_chunks),
      out_type=jax.ShapeDtypeStruct((NC, N_PAD, DEG_W), jnp.float32),
      mesh=_mesh(),
      scratch_types=[
          pltpu.VMEM((n_chunks, CHUNK), jnp.int32),
          pltpu.VMEM((CHUNK, DEG_W), jnp.float32),
          pltpu.VMEM_SHARED((N_PAD, DEG_W), jnp.float32),
          pltpu.SemaphoreType.DMA((2,)),
      ],
      compiler_params=_SC_PARAMS,
  )


def _make_agg_kernel(n_chunks):
  return pl.kernel(
      functools.partial(_agg_body, n_chunks),
      out_type=jax.ShapeDtypeStruct((NC, N_PAD, F_H), jnp.float32),
      mesh=_mesh(),
      scratch_types=[
          pltpu.VMEM((n_chunks, CHUNK), jnp.int32),
          pltpu.VMEM((n_chunks, CHUNK), jnp.int32),
          pltpu.VMEM((3, CHUNK, F_H), jnp.float32),
          pltpu.VMEM_SHARED((N_PAD, F_H), jnp.float32),
          pltpu.VMEM_SHARED((N_PAD, F_H), jnp.float32),
          pltpu.SemaphoreType.DMA((3,)),
          pltpu.SemaphoreType.DMA((3,)),
          pltpu.SemaphoreType.DMA((2,)),
      ],
      compiler_params=_SC_PARAMS,
  )


def _dinv_from(degp):
  deg = degp[0, :, 0:1] + degp[1, :, 0:1] + 1.0
  return lax.rsqrt(deg)


def _tc1_body(x_ref, w_ref, degp_ref, out_ref):
  dinv = _dinv_from(degp_ref[...])
  h = jnp.dot(x_ref[...], w_ref[...], preferred_element_type=jnp.float32)
  out_ref[...] = h * dinv


def _tc2_body(q_ref, hp_ref, degp_ref, b_ref, w_ref, out_ref):
  dinv = _dinv_from(degp_ref[...])
  ssum = q_ref[0] + q_ref[1] + hp_ref[...]
  h = jnp.maximum(ssum * dinv + b_ref[...], 0.0)
  out_ref[...] = jnp.dot(
      h, w_ref[...], preferred_element_type=jnp.float32) * dinv


def _tc3_body(q_ref, hp_ref, degp_ref, b_ref, wfc_ref, bfc_ref, out_ref):
  dinv = _dinv_from(degp_ref[...])
  ssum = q_ref[0] + q_ref[1] + hp_ref[...]
  h = jnp.maximum(ssum * dinv + b_ref[...], 0.0)
  z = jnp.dot(h, wfc_ref[...], preferred_element_type=jnp.float32)
  z = z + bfc_ref[...]
  m = jnp.max(z, axis=1, keepdims=True)
  lse = jnp.log(jnp.sum(jnp.exp(z - m), axis=1, keepdims=True)) + m
  out_ref[...] = z - lse


def _row_spec(w):
  return pl.BlockSpec((BLK, w), lambda i: (i, 0))


def _full_spec(shape):
  nd = len(shape)
  return pl.BlockSpec(shape, lambda i: (0,) * nd)


_DEGP_SPEC = pl.BlockSpec((NC, BLK, DEG_W), lambda i: (0, i, 0))
_Q_SPEC = pl.BlockSpec((NC, BLK, F_H), lambda i: (0, i, 0))
_GRID = (N // BLK,)


def _tc1(x, W1, degp):
  return pl.pallas_call(
      _tc1_body,
      grid=_GRID,
      in_specs=[_row_spec(F_IN), _full_spec((F_IN, F_H)), _DEGP_SPEC],
      out_specs=_row_spec(F_H),
      out_shape=jax.ShapeDtypeStruct((N_PAD, F_H), jnp.float32),
  )(x, W1, degp)


def _tc2(q, hp, degp, b1, W2):
  return pl.pallas_call(
      _tc2_body,
      grid=_GRID,
      in_specs=[_Q_SPEC, _row_spec(F_H), _DEGP_SPEC,
                _full_spec((1, F_H)), _full_spec((F_H, F_H))],
      out_specs=_row_spec(F_H),
      out_shape=jax.ShapeDtypeStruct((N_PAD, F_H), jnp.float32),
  )(q, hp, degp, b1, W2)


def _tc3(q, hp, degp, b2, Wfc, bfc):
  return pl.pallas_call(
      _tc3_body,
      grid=_GRID,
      in_specs=[_Q_SPEC, _row_spec(F_H), _DEGP_SPEC,
                _full_spec((1, F_H)), _full_spec((F_H, F_OUT)),
                _full_spec((1, F_OUT))],
      out_specs=_row_spec(F_OUT),
      out_shape=jax.ShapeDtypeStruct((N, F_OUT), jnp.float32),
  )(q, hp, degp, b2, Wfc, bfc)


def kernel(x, edge_index, W1, b1, W2, b2, Wfc, bfc):
  e = edge_index.shape[1]
  ei = edge_index.astype(jnp.int32)

  per_round = NW * CHUNK
  n_chunks = -(-e // per_round)
  e_pad = n_chunks * per_round
  if e_pad != e:
    fill = jnp.full((2, e_pad - e), N, jnp.int32)
    ei = jnp.concatenate([ei, fill], axis=1)
  eidx = ei.reshape(2, NW, n_chunks, CHUNK)

  zeros_deg = jnp.zeros((N_PAD, DEG_W), jnp.float32)
  zeros_h = jnp.zeros((N_PAD, F_H), jnp.float32)
  ones = jnp.ones((CHUNK, DEG_W), jnp.float32)

  deg_kernel = _make_deg_kernel(n_chunks)
  agg_kernel = _make_agg_kernel(n_chunks)

  degp = deg_kernel(eidx, ones, zeros_deg)
  h1p = _tc1(x, W1, degp)
  q1 = agg_kernel(eidx, h1p, zeros_h)
  h2p = _tc2(q1, h1p, degp, b1[None, :], W2)
  q2 = agg_kernel(eidx, h2p, zeros_h)
  return _tc3(q2, h2p, degp, b2[None, :], Wfc, bfc[None, :])
